# Initial kernel scaffold; baseline (speedup 1.0000x reference)
import jax
import jax.numpy as jnp
from jax.experimental import pallas as pl


def kernel(x, table):
    raise NotImplementedError("write your pallas kernel here")


# speedup vs baseline: 1.1058x; 1.1058x over previous
import functools

import jax
import jax.numpy as jnp
from jax import lax
from jax.experimental import pallas as pl
from jax.experimental.pallas import tpu as pltpu
from jax.experimental.pallas import tpu_sc as plsc

_NC = 2
_NS = 16
_NW = _NC * _NS

_BATCH = 16384
_HIST = 50
_D = 32
_B = _BATCH * _HIST
_BPW = _B // _NW
_C = 1280
_NCHUNK = _BPW // _C


def _embed_body(x_hbm, table_hbm, out_hbm, idx_v, rows_v, gsem):
    wid = lax.axis_index("s") * _NC + lax.axis_index("c")
    base = wid * _BPW
    pltpu.sync_copy(x_hbm.at[pl.ds(base, _BPW)], idx_v)

    @pl.loop(0, _NCHUNK)
    def _chunk(g):
        off = pl.multiple_of(g * _C, _C)
        pltpu.async_copy(
            table_hbm.at[idx_v.at[pl.ds(off, _C)]], rows_v, gsem
        ).wait()
        pltpu.sync_copy(rows_v, out_hbm.at[pl.ds(base + off, _C)])


def kernel(x, table):
    xf = x.reshape(_B)
    mesh = plsc.VectorSubcoreMesh(
        core_axis_name="c", subcore_axis_name="s",
        num_cores=_NC, num_subcores=_NS,
    )
    run = functools.partial(
        pl.kernel,
        out_type=jax.ShapeDtypeStruct((_B, _D), jnp.float32),
        mesh=mesh,
        scratch_types=[
            pltpu.VMEM((_BPW,), jnp.int32),
            pltpu.VMEM((_C, _D), jnp.float32),
            pltpu.SemaphoreType.DMA,
        ],
        compiler_params=pltpu.CompilerParams(use_tc_tiling_on_sc=False),
    )(_embed_body)
    out = run(xf, table)
    return out.reshape(_BATCH, _HIST, _D)


# speedup vs baseline: 1.1142x; 1.0076x over previous
import functools

import jax
import jax.numpy as jnp
from jax import lax
from jax.experimental import pallas as pl
from jax.experimental.pallas import tpu as pltpu
from jax.experimental.pallas import tpu_sc as plsc

_NC = 2
_NS = 16
_NW = _NC * _NS

_BATCH = 16384
_HIST = 50
_D = 32
_B = _BATCH * _HIST
_BPW = _B // _NW
_NBUF = 4
_C = 640
_NCHUNK = _BPW // _C


def _embed_body(x_hbm, table_hbm, out_hbm, idx_v, rows_v, *sems):
    gsems, osems = sems[:_NBUF], sems[_NBUF:]
    wid = lax.axis_index("s") * _NC + lax.axis_index("c")
    base = wid * _BPW
    pltpu.sync_copy(x_hbm.at[pl.ds(base, _BPW)], idx_v)

    def start_gather(c, b):
        off = pl.multiple_of(c * _C, _C)
        pltpu.async_copy(
            table_hbm.at[idx_v.at[pl.ds(off, _C)]], rows_v.at[b], gsems[b]
        )

    def start_write(c, b):
        off = pl.multiple_of(c * _C, _C)
        pltpu.async_copy(rows_v.at[b], out_hbm.at[pl.ds(base + off, _C)],
                         osems[b])

    def drain_gather(b):
        pltpu.make_async_copy(
            table_hbm.at[pl.ds(0, _C)], rows_v.at[b], gsems[b]
        ).wait()

    def drain_write(b):
        pltpu.make_async_copy(
            rows_v.at[b], out_hbm.at[pl.ds(base, _C)], osems[b]
        ).wait()

    for b in range(_NBUF):
        start_gather(b, b)

    @pl.loop(0, _NCHUNK - _NBUF, step=_NBUF)
    def _round(g):
        for b in range(_NBUF):
            c = g + b
            drain_gather(b)
            start_write(c, b)
            drain_write(b)
            start_gather(c + _NBUF, b)

    for b in range(_NBUF):
        drain_gather(b)
        start_write(_NCHUNK - _NBUF + b, b)
    for b in range(_NBUF):
        drain_write(b)


def kernel(x, table):
    xf = x.reshape(_B)
    mesh = plsc.VectorSubcoreMesh(
        core_axis_name="c", subcore_axis_name="s",
        num_cores=_NC, num_subcores=_NS,
    )
    run = functools.partial(
        pl.kernel,
        out_type=jax.ShapeDtypeStruct((_B, _D), jnp.float32),
        mesh=mesh,
        scratch_types=(
            [pltpu.VMEM((_BPW,), jnp.int32),
             pltpu.VMEM((_NBUF, _C, _D), jnp.float32)]
            + [pltpu.SemaphoreType.DMA] * (2 * _NBUF)
        ),
        compiler_params=pltpu.CompilerParams(use_tc_tiling_on_sc=False),
    )(_embed_body)
    out = run(xf, table)
    return out.reshape(_BATCH, _HIST, _D)
